# baseline (speedup 1.0000x reference)
import functools

import numpy as np
import jax
import jax.numpy as jnp
from jax.experimental import pallas as pl
from jax.experimental.pallas import tpu as pltpu

B = 1024
N = 20000
N_NUM = 32
N_FREQ = 48
D_EMB = 64
D_HIDDEN = 256
N_CLASSES = 10

K_GRP = 4
G = N_NUM // K_GRP
ZW = K_GRP * N_FREQ
KOUT = K_GRP * D_EMB
Z_COLS = G * ZW

NB = 2000
N_BLOCKS = N // NB
QB = 512
Q_BLOCKS = B // QB

_EPS = 1e-7
_LOG2E = 1.4426950408889634


_PI2 = np.pi * np.pi


def _sincos_half_turns(t2):
    k = jnp.floor(t2 + 0.5)
    r = t2 - k
    q1 = k - 2.0 * jnp.floor(k * 0.5)
    sg = 1.0 - (q1 + q1)
    rb = r.astype(jnp.bfloat16)
    sgb = sg.astype(jnp.bfloat16)
    u = rb * rb
    bf = jnp.bfloat16
    c = ((u * bf(_PI2**4 / 40320.0) - bf(_PI2**3 / 720.0)) * u
         + bf(_PI2**2 / 24.0)) * u - bf(_PI2 / 2.0)
    c = c * u + bf(1.0)
    s = ((u * bf(-_PI2**3 * np.pi / 5040.0) + bf(_PI2**2 * np.pi / 120.0)) * u
         - bf(_PI2 * np.pi / 6.0)) * u + bf(np.pi)
    s = s * rb
    return sgb * c, sgb * s


def _encode_groups(x_ref, sf_ref, wkc_ref, wks_ref, bk_ref, w1g_ref, b1_ref):
    r = x_ref.shape[0]
    t2 = jnp.dot(x_ref[...].astype(jnp.bfloat16), sf_ref[...],
                 preferred_element_type=jnp.float32)
    cb, sb = _sincos_half_turns(t2)
    acc = jnp.broadcast_to(b1_ref[...], (r, D_HIDDEN))
    for g in range(G):
        h = jnp.dot(cb[:, g * ZW:(g + 1) * ZW], wkc_ref[g],
                    preferred_element_type=jnp.float32)
        h = h + jnp.dot(sb[:, g * ZW:(g + 1) * ZW], wks_ref[g],
                        preferred_element_type=jnp.float32)
        h = jnp.maximum(h + bk_ref[g], 0.0)
        acc = acc + jnp.dot(h.astype(jnp.bfloat16), w1g_ref[g],
                            preferred_element_type=jnp.float32)
    return jnp.maximum(acc, 0.0)


def _enc_body(x_ref, sf_ref, wkc_ref, wks_ref, bk_ref, w1g_ref, b1_ref,
              out_ref):
    acc = _encode_groups(x_ref, sf_ref, wkc_ref, wks_ref, bk_ref, w1g_ref,
                         b1_ref)
    out_ref[...] = acc.astype(jnp.bfloat16)


def _enc_body_t(x_ref, sf_ref, wkc_ref, wks_ref, bk_ref, w1g_ref, b1_ref,
                out_ref):
    acc = _encode_groups(x_ref, sf_ref, wkc_ref, wks_ref, bk_ref, w1g_ref,
                         b1_ref)
    out_ref[...] = (acc * (2.0 * _LOG2E)).T.astype(jnp.bfloat16)


_ENC_IN_SPECS = [
    pl.BlockSpec((N_NUM, Z_COLS), lambda i: (0, 0)),
    pl.BlockSpec((G, ZW, KOUT), lambda i: (0, 0, 0)),
    pl.BlockSpec((G, ZW, KOUT), lambda i: (0, 0, 0)),
    pl.BlockSpec((G, 1, KOUT), lambda i: (0, 0, 0)),
    pl.BlockSpec((G, KOUT, D_HIDDEN), lambda i: (0, 0, 0)),
    pl.BlockSpec((1, D_HIDDEN), lambda i: (0, 0)),
]


def _encode(x, sf, wkc, wks, bk, w1g, b1, rows_per_block):
    rows = x.shape[0]
    grid = (rows // rows_per_block,)
    return pl.pallas_call(
        _enc_body,
        grid=grid,
        in_specs=[pl.BlockSpec((rows_per_block, N_NUM), lambda i: (i, 0))]
        + _ENC_IN_SPECS,
        out_specs=pl.BlockSpec((rows_per_block, D_HIDDEN), lambda i: (i, 0)),
        out_shape=jax.ShapeDtypeStruct((rows, D_HIDDEN), jnp.bfloat16),
        compiler_params=pltpu.CompilerParams(
            dimension_semantics=("parallel",)),
    )(x, sf, wkc, wks, bk, w1g, b1)


def _encode_t(x, sf, wkc, wks, bk, w1g, b1, rows_per_block):
    rows = x.shape[0]
    grid = (rows // rows_per_block,)
    return pl.pallas_call(
        _enc_body_t,
        grid=grid,
        in_specs=[pl.BlockSpec((rows_per_block, N_NUM), lambda i: (i, 0))]
        + _ENC_IN_SPECS,
        out_specs=pl.BlockSpec((D_HIDDEN, rows_per_block), lambda i: (0, i)),
        out_shape=jax.ShapeDtypeStruct((D_HIDDEN, rows), jnp.bfloat16),
        compiler_params=pltpu.CompilerParams(
            dimension_semantics=("parallel",)),
    )(x, sf, wkc, wks, bk, w1g, b1)


def _fused_body(qt_ref, x_ref, sf_ref, wkc_ref, wks_ref, bk_ref, w1g_ref,
                b1_ref, y_ref, out_ref, m_ref, acc_ref):
    nb = pl.program_id(0)
    enc = _encode_groups(x_ref, sf_ref, wkc_ref, wks_ref, bk_ref, w1g_ref,
                         b1_ref)
    c2 = jnp.sum(enc * enc, axis=1, keepdims=True)
    s = jnp.dot(enc.astype(jnp.bfloat16), qt_ref[...],
                preferred_element_type=jnp.float32) - _LOG2E * c2
    bm = jnp.max(s, axis=0, keepdims=True)
    m_prev = jnp.where(nb == 0, jnp.full_like(bm, -1e30), m_ref[0:1, :])
    m_new = jnp.maximum(m_prev, bm)
    e = jnp.exp2((s - m_new).astype(jnp.bfloat16))
    yrow = jnp.broadcast_to(y_ref[0], (16, NB))
    ridx = jax.lax.broadcasted_iota(jnp.int32, (16, NB), 0)
    ya = jnp.logical_or(ridx == yrow, ridx == N_CLASSES).astype(jnp.bfloat16)
    p = jnp.dot(ya, e, preferred_element_type=jnp.float32)
    scale = jnp.exp2(m_prev - m_new)
    acc_prev = jnp.where(nb == 0, jnp.zeros_like(acc_ref[...]), acc_ref[...])
    acc = acc_prev * scale + p
    m_ref[0:1, :] = m_new
    acc_ref[...] = acc

    @pl.when(nb == N_BLOCKS - 1)
    def _():
        denom = acc[N_CLASSES:N_CLASSES + 1, :]
        res = jnp.log(acc / denom + _EPS)
        ridx2 = jax.lax.broadcasted_iota(jnp.int32, res.shape, 0)
        out_ref[...] = jnp.where(ridx2 < N_CLASSES, res, 0.0)


def _fused(qt, x, sf, wkc, wks, bk, w1g, b1, y3):
    return pl.pallas_call(
        _fused_body,
        grid=(N_BLOCKS,),
        in_specs=[pl.BlockSpec((D_HIDDEN, B), lambda i: (0, 0)),
                  pl.BlockSpec((NB, N_NUM), lambda i: (i, 0))]
        + _ENC_IN_SPECS
        + [pl.BlockSpec((1, 1, NB), lambda i: (i, 0, 0))],
        out_specs=pl.BlockSpec((16, B), lambda i: (0, 0)),
        out_shape=jax.ShapeDtypeStruct((16, B), jnp.float32),
        scratch_shapes=[
            pltpu.VMEM((8, B), jnp.float32),
            pltpu.VMEM((16, B), jnp.float32),
        ],
        compiler_params=pltpu.CompilerParams(
            dimension_semantics=("arbitrary",)),
    )(qt, x, sf, wkc, wks, bk, w1g, b1, y3)


def _dist_body(qt_ref, c_ref, y_ref, out_ref, m_ref, acc_ref):
    nb = pl.program_id(1)
    c = c_ref[...]
    cf = c.astype(jnp.float32)
    c2 = jnp.sum(cf * cf, axis=1, keepdims=True)
    s = jnp.dot(c, qt_ref[...],
                preferred_element_type=jnp.float32) - _LOG2E * c2
    bm = jnp.max(s, axis=0, keepdims=True)
    m_prev = jnp.where(nb == 0, jnp.full_like(bm, -1e30), m_ref[0:1, :])
    m_new = jnp.maximum(m_prev, bm)
    e = jnp.exp2((s - m_new).astype(jnp.bfloat16))
    yrow = jnp.broadcast_to(y_ref[0], (16, NB))
    ridx = jax.lax.broadcasted_iota(jnp.int32, (16, NB), 0)
    ya = jnp.logical_or(ridx == yrow, ridx == N_CLASSES).astype(jnp.bfloat16)
    p = jnp.dot(ya, e, preferred_element_type=jnp.float32)
    scale = jnp.exp2(m_prev - m_new)
    acc_prev = jnp.where(nb == 0, jnp.zeros_like(acc_ref[...]), acc_ref[...])
    acc = acc_prev * scale + p
    m_ref[0:1, :] = m_new
    acc_ref[...] = acc

    @pl.when(nb == N_BLOCKS - 1)
    def _():
        denom = acc[N_CLASSES:N_CLASSES + 1, :]
        res = jnp.log(acc / denom + _EPS)
        ridx2 = jax.lax.broadcasted_iota(jnp.int32, res.shape, 0)
        out_ref[...] = jnp.where(ridx2 < N_CLASSES, res, 0.0)


def _distance(qt, cenc, y3):
    return pl.pallas_call(
        _dist_body,
        grid=(Q_BLOCKS, N_BLOCKS),
        in_specs=[
            pl.BlockSpec((D_HIDDEN, QB), lambda qb, nb: (0, qb)),
            pl.BlockSpec((NB, D_HIDDEN), lambda qb, nb: (nb, 0)),
            pl.BlockSpec((1, 1, NB), lambda qb, nb: (nb, 0, 0)),
        ],
        out_specs=pl.BlockSpec((16, QB), lambda qb, nb: (0, qb)),
        out_shape=jax.ShapeDtypeStruct((16, B), jnp.float32),
        scratch_shapes=[
            pltpu.VMEM((8, QB), jnp.float32),
            pltpu.VMEM((16, QB), jnp.float32),
        ],
        compiler_params=pltpu.CompilerParams(
            dimension_semantics=("parallel", "arbitrary")),
    )(qt, cenc, y3)


def kernel(x_num, candidate_x_num, candidate_y, freq, W_enc, b_enc, W1, b1):
    f32 = jnp.float32
    freq = freq.astype(f32)
    n_idx = jnp.arange(N_NUM)
    cols = (n_idx * N_FREQ)[:, None] + jnp.arange(N_FREQ)[None, :]
    sf = jnp.zeros((N_NUM, Z_COLS), f32)
    sf = sf.at[n_idx[:, None], cols].set(2.0 * freq)
    sf = sf.astype(jnp.bfloat16)

    we = W_enc.astype(f32).reshape(G, K_GRP, 2, N_FREQ, D_EMB)
    wkc = jnp.zeros((G, ZW, KOUT), f32)
    wks = jnp.zeros((G, ZW, KOUT), f32)
    for j in range(K_GRP):
        rr = j * N_FREQ
        cc = j * D_EMB
        wkc = wkc.at[:, rr:rr + N_FREQ, cc:cc + D_EMB].set(we[:, j, 0])
        wks = wks.at[:, rr:rr + N_FREQ, cc:cc + D_EMB].set(we[:, j, 1])
    wkc = wkc.astype(jnp.bfloat16)
    wks = wks.astype(jnp.bfloat16)

    bk = b_enc.astype(f32).reshape(G, 1, KOUT)
    w1g = W1.astype(f32).reshape(G, KOUT, D_HIDDEN).astype(jnp.bfloat16)
    b1r = b1.astype(f32).reshape(1, D_HIDDEN)

    qt = _encode_t(x_num.astype(f32), sf, wkc, wks, bk, w1g, b1r, B // 2)
    y3 = candidate_y.astype(jnp.int32).reshape(N_BLOCKS, 1, NB)
    out = _fused(qt, candidate_x_num.astype(f32), sf, wkc, wks, bk, w1g,
                 b1r, y3)
    return out[:N_CLASSES, :].T


# speedup vs baseline: 1.0355x; 1.0355x over previous
import functools

import numpy as np
import jax
import jax.numpy as jnp
from jax.experimental import pallas as pl
from jax.experimental.pallas import tpu as pltpu

B = 1024
N = 20000
N_NUM = 32
N_FREQ = 48
D_EMB = 64
D_HIDDEN = 256
N_CLASSES = 10

K_GRP = 4
G = N_NUM // K_GRP
ZW = K_GRP * N_FREQ
KOUT = K_GRP * D_EMB
Z_COLS = G * ZW

NB = 2000
N_BLOCKS = N // NB
QB = 512
Q_BLOCKS = B // QB

_EPS = 1e-7
_LOG2E = 1.4426950408889634


_PI2 = np.pi * np.pi


def _sincos_half_turns(t2):
    big = jnp.float32(12582912.0)
    y = t2 + big
    k = y - big
    r = t2 - k
    yi = jax.lax.bitcast_convert_type(y, jnp.int32)
    sgi = 1 - ((yi & 1) << 1)
    rb = r.astype(jnp.bfloat16)
    sgb = sgi.astype(jnp.bfloat16)
    u = rb * rb
    bf = jnp.bfloat16
    c = ((u * bf(_PI2**4 / 40320.0) - bf(_PI2**3 / 720.0)) * u
         + bf(_PI2**2 / 24.0)) * u - bf(_PI2 / 2.0)
    c = c * u + bf(1.0)
    s = ((u * bf(-_PI2**3 * np.pi / 5040.0) + bf(_PI2**2 * np.pi / 120.0)) * u
         - bf(_PI2 * np.pi / 6.0)) * u + bf(np.pi)
    s = s * rb
    return sgb * c, sgb * s


def _encode_groups(x_ref, sf_ref, wkc_ref, wks_ref, bk_ref, w1g_ref, b1_ref):
    r = x_ref.shape[0]
    t2 = jnp.dot(x_ref[...].astype(jnp.bfloat16), sf_ref[...],
                 preferred_element_type=jnp.float32)
    cb, sb = _sincos_half_turns(t2)
    acc = jnp.broadcast_to(b1_ref[...], (r, D_HIDDEN))
    for g in range(G):
        h = jnp.dot(cb[:, g * ZW:(g + 1) * ZW], wkc_ref[g],
                    preferred_element_type=jnp.float32)
        h = h + jnp.dot(sb[:, g * ZW:(g + 1) * ZW], wks_ref[g],
                        preferred_element_type=jnp.float32)
        h = jnp.maximum(h + bk_ref[g], 0.0)
        acc = acc + jnp.dot(h.astype(jnp.bfloat16), w1g_ref[g],
                            preferred_element_type=jnp.float32)
    return jnp.maximum(acc, 0.0)


def _enc_body(x_ref, sf_ref, wkc_ref, wks_ref, bk_ref, w1g_ref, b1_ref,
              out_ref):
    acc = _encode_groups(x_ref, sf_ref, wkc_ref, wks_ref, bk_ref, w1g_ref,
                         b1_ref)
    out_ref[...] = acc.astype(jnp.bfloat16)


def _enc_body_t(x_ref, sf_ref, wkc_ref, wks_ref, bk_ref, w1g_ref, b1_ref,
                out_ref):
    acc = _encode_groups(x_ref, sf_ref, wkc_ref, wks_ref, bk_ref, w1g_ref,
                         b1_ref)
    out_ref[...] = (acc * (2.0 * _LOG2E)).T.astype(jnp.bfloat16)


_ENC_IN_SPECS = [
    pl.BlockSpec((N_NUM, Z_COLS), lambda i: (0, 0)),
    pl.BlockSpec((G, ZW, KOUT), lambda i: (0, 0, 0)),
    pl.BlockSpec((G, ZW, KOUT), lambda i: (0, 0, 0)),
    pl.BlockSpec((G, 1, KOUT), lambda i: (0, 0, 0)),
    pl.BlockSpec((G, KOUT, D_HIDDEN), lambda i: (0, 0, 0)),
    pl.BlockSpec((1, D_HIDDEN), lambda i: (0, 0)),
]


def _encode(x, sf, wkc, wks, bk, w1g, b1, rows_per_block):
    rows = x.shape[0]
    grid = (rows // rows_per_block,)
    return pl.pallas_call(
        _enc_body,
        grid=grid,
        in_specs=[pl.BlockSpec((rows_per_block, N_NUM), lambda i: (i, 0))]
        + _ENC_IN_SPECS,
        out_specs=pl.BlockSpec((rows_per_block, D_HIDDEN), lambda i: (i, 0)),
        out_shape=jax.ShapeDtypeStruct((rows, D_HIDDEN), jnp.bfloat16),
        compiler_params=pltpu.CompilerParams(
            dimension_semantics=("parallel",)),
    )(x, sf, wkc, wks, bk, w1g, b1)


def _encode_t(x, sf, wkc, wks, bk, w1g, b1, rows_per_block):
    rows = x.shape[0]
    grid = (rows // rows_per_block,)
    return pl.pallas_call(
        _enc_body_t,
        grid=grid,
        in_specs=[pl.BlockSpec((rows_per_block, N_NUM), lambda i: (i, 0))]
        + _ENC_IN_SPECS,
        out_specs=pl.BlockSpec((D_HIDDEN, rows_per_block), lambda i: (0, i)),
        out_shape=jax.ShapeDtypeStruct((D_HIDDEN, rows), jnp.bfloat16),
        compiler_params=pltpu.CompilerParams(
            dimension_semantics=("parallel",)),
    )(x, sf, wkc, wks, bk, w1g, b1)


def _fused_body(qt_ref, x_ref, sf_ref, wkc_ref, wks_ref, bk_ref, w1g_ref,
                b1_ref, y_ref, out_ref, m_ref, acc_ref):
    nb = pl.program_id(0)
    enc = _encode_groups(x_ref, sf_ref, wkc_ref, wks_ref, bk_ref, w1g_ref,
                         b1_ref)
    c2 = jnp.sum(enc * enc, axis=1, keepdims=True)
    s = jnp.dot(enc.astype(jnp.bfloat16), qt_ref[...],
                preferred_element_type=jnp.float32) - _LOG2E * c2
    bm = jnp.max(s, axis=0, keepdims=True)
    m_prev = jnp.where(nb == 0, jnp.full_like(bm, -1e30), m_ref[0:1, :])
    m_new = jnp.maximum(m_prev, bm)
    e = jnp.exp2((s - m_new).astype(jnp.bfloat16))
    yrow = jnp.broadcast_to(y_ref[0], (16, NB))
    ridx = jax.lax.broadcasted_iota(jnp.int32, (16, NB), 0)
    ya = jnp.logical_or(ridx == yrow, ridx == N_CLASSES).astype(jnp.bfloat16)
    p = jnp.dot(ya, e, preferred_element_type=jnp.float32)
    scale = jnp.exp2(m_prev - m_new)
    acc_prev = jnp.where(nb == 0, jnp.zeros_like(acc_ref[...]), acc_ref[...])
    acc = acc_prev * scale + p
    m_ref[0:1, :] = m_new
    acc_ref[...] = acc

    @pl.when(nb == N_BLOCKS - 1)
    def _():
        denom = acc[N_CLASSES:N_CLASSES + 1, :]
        res = jnp.log(acc / denom + _EPS)
        ridx2 = jax.lax.broadcasted_iota(jnp.int32, res.shape, 0)
        out_ref[...] = jnp.where(ridx2 < N_CLASSES, res, 0.0)


def _fused(qt, x, sf, wkc, wks, bk, w1g, b1, y3):
    return pl.pallas_call(
        _fused_body,
        grid=(N_BLOCKS,),
        in_specs=[pl.BlockSpec((D_HIDDEN, B), lambda i: (0, 0)),
                  pl.BlockSpec((NB, N_NUM), lambda i: (i, 0))]
        + _ENC_IN_SPECS
        + [pl.BlockSpec((1, 1, NB), lambda i: (i, 0, 0))],
        out_specs=pl.BlockSpec((16, B), lambda i: (0, 0)),
        out_shape=jax.ShapeDtypeStruct((16, B), jnp.float32),
        scratch_shapes=[
            pltpu.VMEM((8, B), jnp.float32),
            pltpu.VMEM((16, B), jnp.float32),
        ],
        compiler_params=pltpu.CompilerParams(
            dimension_semantics=("arbitrary",)),
    )(qt, x, sf, wkc, wks, bk, w1g, b1, y3)


def _dist_body(qt_ref, c_ref, y_ref, out_ref, m_ref, acc_ref):
    nb = pl.program_id(1)
    c = c_ref[...]
    cf = c.astype(jnp.float32)
    c2 = jnp.sum(cf * cf, axis=1, keepdims=True)
    s = jnp.dot(c, qt_ref[...],
                preferred_element_type=jnp.float32) - _LOG2E * c2
    bm = jnp.max(s, axis=0, keepdims=True)
    m_prev = jnp.where(nb == 0, jnp.full_like(bm, -1e30), m_ref[0:1, :])
    m_new = jnp.maximum(m_prev, bm)
    e = jnp.exp2((s - m_new).astype(jnp.bfloat16))
    yrow = jnp.broadcast_to(y_ref[0], (16, NB))
    ridx = jax.lax.broadcasted_iota(jnp.int32, (16, NB), 0)
    ya = jnp.logical_or(ridx == yrow, ridx == N_CLASSES).astype(jnp.bfloat16)
    p = jnp.dot(ya, e, preferred_element_type=jnp.float32)
    scale = jnp.exp2(m_prev - m_new)
    acc_prev = jnp.where(nb == 0, jnp.zeros_like(acc_ref[...]), acc_ref[...])
    acc = acc_prev * scale + p
    m_ref[0:1, :] = m_new
    acc_ref[...] = acc

    @pl.when(nb == N_BLOCKS - 1)
    def _():
        denom = acc[N_CLASSES:N_CLASSES + 1, :]
        res = jnp.log(acc / denom + _EPS)
        ridx2 = jax.lax.broadcasted_iota(jnp.int32, res.shape, 0)
        out_ref[...] = jnp.where(ridx2 < N_CLASSES, res, 0.0)


def _distance(qt, cenc, y3):
    return pl.pallas_call(
        _dist_body,
        grid=(Q_BLOCKS, N_BLOCKS),
        in_specs=[
            pl.BlockSpec((D_HIDDEN, QB), lambda qb, nb: (0, qb)),
            pl.BlockSpec((NB, D_HIDDEN), lambda qb, nb: (nb, 0)),
            pl.BlockSpec((1, 1, NB), lambda qb, nb: (nb, 0, 0)),
        ],
        out_specs=pl.BlockSpec((16, QB), lambda qb, nb: (0, qb)),
        out_shape=jax.ShapeDtypeStruct((16, B), jnp.float32),
        scratch_shapes=[
            pltpu.VMEM((8, QB), jnp.float32),
            pltpu.VMEM((16, QB), jnp.float32),
        ],
        compiler_params=pltpu.CompilerParams(
            dimension_semantics=("parallel", "arbitrary")),
    )(qt, cenc, y3)


def kernel(x_num, candidate_x_num, candidate_y, freq, W_enc, b_enc, W1, b1):
    f32 = jnp.float32
    freq = freq.astype(f32)
    n_idx = jnp.arange(N_NUM)
    cols = (n_idx * N_FREQ)[:, None] + jnp.arange(N_FREQ)[None, :]
    sf = jnp.zeros((N_NUM, Z_COLS), f32)
    sf = sf.at[n_idx[:, None], cols].set(2.0 * freq)
    sf = sf.astype(jnp.bfloat16)

    we = W_enc.astype(f32).reshape(G, K_GRP, 2, N_FREQ, D_EMB)
    wkc = jnp.zeros((G, ZW, KOUT), f32)
    wks = jnp.zeros((G, ZW, KOUT), f32)
    for j in range(K_GRP):
        rr = j * N_FREQ
        cc = j * D_EMB
        wkc = wkc.at[:, rr:rr + N_FREQ, cc:cc + D_EMB].set(we[:, j, 0])
        wks = wks.at[:, rr:rr + N_FREQ, cc:cc + D_EMB].set(we[:, j, 1])
    wkc = wkc.astype(jnp.bfloat16)
    wks = wks.astype(jnp.bfloat16)

    bk = b_enc.astype(f32).reshape(G, 1, KOUT)
    w1g = W1.astype(f32).reshape(G, KOUT, D_HIDDEN).astype(jnp.bfloat16)
    b1r = b1.astype(f32).reshape(1, D_HIDDEN)

    qt = _encode_t(x_num.astype(f32), sf, wkc, wks, bk, w1g, b1r, B // 2)
    y3 = candidate_y.astype(jnp.int32).reshape(N_BLOCKS, 1, NB)
    out = _fused(qt, candidate_x_num.astype(f32), sf, wkc, wks, bk, w1g,
                 b1r, y3)
    return out[:N_CLASSES, :].T


# speedup vs baseline: 1.0762x; 1.0393x over previous
import functools

import numpy as np
import jax
import jax.numpy as jnp
from jax.experimental import pallas as pl
from jax.experimental.pallas import tpu as pltpu

B = 1024
N = 20000
N_NUM = 32
N_FREQ = 48
D_EMB = 64
D_HIDDEN = 256
N_CLASSES = 10

K_GRP = 4
G = N_NUM // K_GRP
ZW = K_GRP * N_FREQ
KOUT = K_GRP * D_EMB
Z_COLS = G * ZW

NB = 2000
N_BLOCKS = N // NB
QB = 512
Q_BLOCKS = B // QB

_EPS = 1e-7
_LOG2E = 1.4426950408889634


_PI2 = np.pi * np.pi


def _sincos_half_turns(t2):
    big = jnp.float32(12582912.0)
    y = t2 + big
    k = y - big
    r = t2 - k
    yi = jax.lax.bitcast_convert_type(y, jnp.int32)
    sgi = 1 - ((yi & 1) << 1)
    rb = r.astype(jnp.bfloat16)
    sgb = sgi.astype(jnp.bfloat16)
    u = rb * rb
    bf = jnp.bfloat16
    c = ((bf(-_PI2**3 / 720.0) * u + bf(_PI2**2 / 24.0)) * u
         - bf(_PI2 / 2.0)) * u + bf(1.0)
    s = ((bf(_PI2**2 * np.pi / 120.0) * u - bf(_PI2 * np.pi / 6.0)) * u
         + bf(np.pi)) * rb
    return sgb * c, sgb * s


def _encode_groups(x_ref, sf_ref, wkc_ref, wks_ref, bk_ref, w1g_ref, b1_ref):
    r = x_ref.shape[0]
    t2 = jnp.dot(x_ref[...].astype(jnp.bfloat16), sf_ref[...],
                 preferred_element_type=jnp.float32)
    cb, sb = _sincos_half_turns(t2)
    acc = jnp.broadcast_to(b1_ref[...], (r, D_HIDDEN))
    for g in range(G):
        h = jnp.dot(cb[:, g * ZW:(g + 1) * ZW], wkc_ref[g],
                    preferred_element_type=jnp.float32)
        h = h + jnp.dot(sb[:, g * ZW:(g + 1) * ZW], wks_ref[g],
                        preferred_element_type=jnp.float32)
        h = jnp.maximum(h + bk_ref[g], 0.0)
        acc = acc + jnp.dot(h.astype(jnp.bfloat16), w1g_ref[g],
                            preferred_element_type=jnp.float32)
    return jnp.maximum(acc, 0.0)


def _enc_body(x_ref, sf_ref, wkc_ref, wks_ref, bk_ref, w1g_ref, b1_ref,
              out_ref):
    acc = _encode_groups(x_ref, sf_ref, wkc_ref, wks_ref, bk_ref, w1g_ref,
                         b1_ref)
    out_ref[...] = acc.astype(jnp.bfloat16)


def _enc_body_t(x_ref, sf_ref, wkc_ref, wks_ref, bk_ref, w1g_ref, b1_ref,
                out_ref):
    acc = _encode_groups(x_ref, sf_ref, wkc_ref, wks_ref, bk_ref, w1g_ref,
                         b1_ref)
    out_ref[...] = (acc * (2.0 * _LOG2E)).T.astype(jnp.bfloat16)


_ENC_IN_SPECS = [
    pl.BlockSpec((N_NUM, Z_COLS), lambda i: (0, 0)),
    pl.BlockSpec((G, ZW, KOUT), lambda i: (0, 0, 0)),
    pl.BlockSpec((G, ZW, KOUT), lambda i: (0, 0, 0)),
    pl.BlockSpec((G, 1, KOUT), lambda i: (0, 0, 0)),
    pl.BlockSpec((G, KOUT, D_HIDDEN), lambda i: (0, 0, 0)),
    pl.BlockSpec((1, D_HIDDEN), lambda i: (0, 0)),
]


def _encode(x, sf, wkc, wks, bk, w1g, b1, rows_per_block):
    rows = x.shape[0]
    grid = (rows // rows_per_block,)
    return pl.pallas_call(
        _enc_body,
        grid=grid,
        in_specs=[pl.BlockSpec((rows_per_block, N_NUM), lambda i: (i, 0))]
        + _ENC_IN_SPECS,
        out_specs=pl.BlockSpec((rows_per_block, D_HIDDEN), lambda i: (i, 0)),
        out_shape=jax.ShapeDtypeStruct((rows, D_HIDDEN), jnp.bfloat16),
        compiler_params=pltpu.CompilerParams(
            dimension_semantics=("parallel",)),
    )(x, sf, wkc, wks, bk, w1g, b1)


def _encode_t(x, sf, wkc, wks, bk, w1g, b1, rows_per_block):
    rows = x.shape[0]
    grid = (rows // rows_per_block,)
    return pl.pallas_call(
        _enc_body_t,
        grid=grid,
        in_specs=[pl.BlockSpec((rows_per_block, N_NUM), lambda i: (i, 0))]
        + _ENC_IN_SPECS,
        out_specs=pl.BlockSpec((D_HIDDEN, rows_per_block), lambda i: (0, i)),
        out_shape=jax.ShapeDtypeStruct((D_HIDDEN, rows), jnp.bfloat16),
        compiler_params=pltpu.CompilerParams(
            dimension_semantics=("parallel",)),
    )(x, sf, wkc, wks, bk, w1g, b1)


def _fused_body(qt_ref, x_ref, sf_ref, wkc_ref, wks_ref, bk_ref, w1g_ref,
                b1_ref, y_ref, out_ref, m_ref, acc_ref):
    nb = pl.program_id(0)
    enc = _encode_groups(x_ref, sf_ref, wkc_ref, wks_ref, bk_ref, w1g_ref,
                         b1_ref)
    c2 = jnp.sum(enc * enc, axis=1, keepdims=True)
    s = jnp.dot(enc.astype(jnp.bfloat16), qt_ref[...],
                preferred_element_type=jnp.float32) - _LOG2E * c2
    bm = jnp.max(s, axis=0, keepdims=True)
    m_prev = jnp.where(nb == 0, jnp.full_like(bm, -1e30), m_ref[0:1, :])
    m_new = jnp.maximum(m_prev, bm)
    e = jnp.exp2((s - m_new).astype(jnp.bfloat16))
    yrow = jnp.broadcast_to(y_ref[0], (16, NB))
    ridx = jax.lax.broadcasted_iota(jnp.int32, (16, NB), 0)
    ya = jnp.logical_or(ridx == yrow, ridx == N_CLASSES).astype(jnp.bfloat16)
    p = jnp.dot(ya, e, preferred_element_type=jnp.float32)
    scale = jnp.exp2(m_prev - m_new)
    acc_prev = jnp.where(nb == 0, jnp.zeros_like(acc_ref[...]), acc_ref[...])
    acc = acc_prev * scale + p
    m_ref[0:1, :] = m_new
    acc_ref[...] = acc

    @pl.when(nb == N_BLOCKS - 1)
    def _():
        denom = acc[N_CLASSES:N_CLASSES + 1, :]
        res = jnp.log(acc / denom + _EPS)
        ridx2 = jax.lax.broadcasted_iota(jnp.int32, res.shape, 0)
        out_ref[...] = jnp.where(ridx2 < N_CLASSES, res, 0.0)


def _fused(qt, x, sf, wkc, wks, bk, w1g, b1, y3):
    return pl.pallas_call(
        _fused_body,
        grid=(N_BLOCKS,),
        in_specs=[pl.BlockSpec((D_HIDDEN, B), lambda i: (0, 0)),
                  pl.BlockSpec((NB, N_NUM), lambda i: (i, 0))]
        + _ENC_IN_SPECS
        + [pl.BlockSpec((1, 1, NB), lambda i: (i, 0, 0))],
        out_specs=pl.BlockSpec((16, B), lambda i: (0, 0)),
        out_shape=jax.ShapeDtypeStruct((16, B), jnp.float32),
        scratch_shapes=[
            pltpu.VMEM((8, B), jnp.float32),
            pltpu.VMEM((16, B), jnp.float32),
        ],
        compiler_params=pltpu.CompilerParams(
            dimension_semantics=("arbitrary",)),
    )(qt, x, sf, wkc, wks, bk, w1g, b1, y3)


def _dist_body(qt_ref, c_ref, y_ref, out_ref, m_ref, acc_ref):
    nb = pl.program_id(1)
    c = c_ref[...]
    cf = c.astype(jnp.float32)
    c2 = jnp.sum(cf * cf, axis=1, keepdims=True)
    s = jnp.dot(c, qt_ref[...],
                preferred_element_type=jnp.float32) - _LOG2E * c2
    bm = jnp.max(s, axis=0, keepdims=True)
    m_prev = jnp.where(nb == 0, jnp.full_like(bm, -1e30), m_ref[0:1, :])
    m_new = jnp.maximum(m_prev, bm)
    e = jnp.exp2((s - m_new).astype(jnp.bfloat16))
    yrow = jnp.broadcast_to(y_ref[0], (16, NB))
    ridx = jax.lax.broadcasted_iota(jnp.int32, (16, NB), 0)
    ya = jnp.logical_or(ridx == yrow, ridx == N_CLASSES).astype(jnp.bfloat16)
    p = jnp.dot(ya, e, preferred_element_type=jnp.float32)
    scale = jnp.exp2(m_prev - m_new)
    acc_prev = jnp.where(nb == 0, jnp.zeros_like(acc_ref[...]), acc_ref[...])
    acc = acc_prev * scale + p
    m_ref[0:1, :] = m_new
    acc_ref[...] = acc

    @pl.when(nb == N_BLOCKS - 1)
    def _():
        denom = acc[N_CLASSES:N_CLASSES + 1, :]
        res = jnp.log(acc / denom + _EPS)
        ridx2 = jax.lax.broadcasted_iota(jnp.int32, res.shape, 0)
        out_ref[...] = jnp.where(ridx2 < N_CLASSES, res, 0.0)


def _distance(qt, cenc, y3):
    return pl.pallas_call(
        _dist_body,
        grid=(Q_BLOCKS, N_BLOCKS),
        in_specs=[
            pl.BlockSpec((D_HIDDEN, QB), lambda qb, nb: (0, qb)),
            pl.BlockSpec((NB, D_HIDDEN), lambda qb, nb: (nb, 0)),
            pl.BlockSpec((1, 1, NB), lambda qb, nb: (nb, 0, 0)),
        ],
        out_specs=pl.BlockSpec((16, QB), lambda qb, nb: (0, qb)),
        out_shape=jax.ShapeDtypeStruct((16, B), jnp.float32),
        scratch_shapes=[
            pltpu.VMEM((8, QB), jnp.float32),
            pltpu.VMEM((16, QB), jnp.float32),
        ],
        compiler_params=pltpu.CompilerParams(
            dimension_semantics=("parallel", "arbitrary")),
    )(qt, cenc, y3)


def kernel(x_num, candidate_x_num, candidate_y, freq, W_enc, b_enc, W1, b1):
    f32 = jnp.float32
    freq = freq.astype(f32)
    n_idx = jnp.arange(N_NUM)
    cols = (n_idx * N_FREQ)[:, None] + jnp.arange(N_FREQ)[None, :]
    sf = jnp.zeros((N_NUM, Z_COLS), f32)
    sf = sf.at[n_idx[:, None], cols].set(2.0 * freq)
    sf = sf.astype(jnp.bfloat16)

    we = W_enc.astype(f32).reshape(G, K_GRP, 2, N_FREQ, D_EMB)
    wkc = jnp.zeros((G, ZW, KOUT), f32)
    wks = jnp.zeros((G, ZW, KOUT), f32)
    for j in range(K_GRP):
        rr = j * N_FREQ
        cc = j * D_EMB
        wkc = wkc.at[:, rr:rr + N_FREQ, cc:cc + D_EMB].set(we[:, j, 0])
        wks = wks.at[:, rr:rr + N_FREQ, cc:cc + D_EMB].set(we[:, j, 1])
    wkc = wkc.astype(jnp.bfloat16)
    wks = wks.astype(jnp.bfloat16)

    bk = b_enc.astype(f32).reshape(G, 1, KOUT)
    w1g = W1.astype(f32).reshape(G, KOUT, D_HIDDEN).astype(jnp.bfloat16)
    b1r = b1.astype(f32).reshape(1, D_HIDDEN)

    qt = _encode_t(x_num.astype(f32), sf, wkc, wks, bk, w1g, b1r, B // 2)
    y3 = candidate_y.astype(jnp.int32).reshape(N_BLOCKS, 1, NB)
    out = _fused(qt, candidate_x_num.astype(f32), sf, wkc, wks, bk, w1g,
                 b1r, y3)
    return out[:N_CLASSES, :].T


# speedup vs baseline: 1.0766x; 1.0004x over previous
import functools

import numpy as np
import jax
import jax.numpy as jnp
from jax.experimental import pallas as pl
from jax.experimental.pallas import tpu as pltpu

B = 1024
N = 20000
N_NUM = 32
N_FREQ = 48
D_EMB = 64
D_HIDDEN = 256
N_CLASSES = 10

K_GRP = 4
G = N_NUM // K_GRP
ZW = K_GRP * N_FREQ
KOUT = K_GRP * D_EMB
Z_COLS = G * ZW

NB = 2000
N_BLOCKS = N // NB

_EPS = 1e-7
_LOG2E = 1.4426950408889634


_PI2 = np.pi * np.pi


def _sincos_half_turns(t2):
    big = jnp.float32(12582912.0)
    y = t2 + big
    k = y - big
    r = t2 - k
    yi = jax.lax.bitcast_convert_type(y, jnp.int32)
    sgi = 1 - ((yi & 1) << 1)
    rb = r.astype(jnp.bfloat16)
    sgb = sgi.astype(jnp.bfloat16)
    u = rb * rb
    bf = jnp.bfloat16
    c = ((bf(-_PI2**3 / 720.0) * u + bf(_PI2**2 / 24.0)) * u
         - bf(_PI2 / 2.0)) * u + bf(1.0)
    s = ((bf(_PI2**2 * np.pi / 120.0) * u - bf(_PI2 * np.pi / 6.0)) * u
         + bf(np.pi)) * rb
    return sgb * c, sgb * s


def _encode_groups(x_ref, sf_ref, wkc_ref, wks_ref, bk_ref, w1g_ref, b1_ref):
    r = x_ref.shape[0]
    t2 = jnp.dot(x_ref[...].astype(jnp.bfloat16), sf_ref[...],
                 preferred_element_type=jnp.float32)
    cb, sb = _sincos_half_turns(t2)
    acc = jnp.broadcast_to(b1_ref[...], (r, D_HIDDEN))
    for g in range(G):
        h = jnp.dot(cb[:, g * ZW:(g + 1) * ZW], wkc_ref[g],
                    preferred_element_type=jnp.float32)
        h = h + jnp.dot(sb[:, g * ZW:(g + 1) * ZW], wks_ref[g],
                        preferred_element_type=jnp.float32)
        h = jnp.maximum(h + bk_ref[g], 0.0)
        acc = acc + jnp.dot(h.astype(jnp.bfloat16), w1g_ref[g],
                            preferred_element_type=jnp.float32)
    return jnp.maximum(acc, 0.0)


def _enc_body_t(x_ref, sf_ref, wkc_ref, wks_ref, bk_ref, w1g_ref, b1_ref,
                out_ref):
    acc = _encode_groups(x_ref, sf_ref, wkc_ref, wks_ref, bk_ref, w1g_ref,
                         b1_ref)
    out_ref[...] = (acc * (2.0 * _LOG2E)).T.astype(jnp.bfloat16)


_ENC_IN_SPECS = [
    pl.BlockSpec((N_NUM, Z_COLS), lambda i: (0, 0)),
    pl.BlockSpec((G, ZW, KOUT), lambda i: (0, 0, 0)),
    pl.BlockSpec((G, ZW, KOUT), lambda i: (0, 0, 0)),
    pl.BlockSpec((G, 1, KOUT), lambda i: (0, 0, 0)),
    pl.BlockSpec((G, KOUT, D_HIDDEN), lambda i: (0, 0, 0)),
    pl.BlockSpec((1, D_HIDDEN), lambda i: (0, 0)),
]


def _encode_t(x, sf, wkc, wks, bk, w1g, b1, rows_per_block):
    rows = x.shape[0]
    grid = (rows // rows_per_block,)
    return pl.pallas_call(
        _enc_body_t,
        grid=grid,
        in_specs=[pl.BlockSpec((rows_per_block, N_NUM), lambda i: (i, 0))]
        + _ENC_IN_SPECS,
        out_specs=pl.BlockSpec((D_HIDDEN, rows_per_block), lambda i: (0, i)),
        out_shape=jax.ShapeDtypeStruct((D_HIDDEN, rows), jnp.bfloat16),
        compiler_params=pltpu.CompilerParams(
            dimension_semantics=("parallel",)),
    )(x, sf, wkc, wks, bk, w1g, b1)


def _fused_body(qt_ref, x_ref, sf_ref, wkc_ref, wks_ref, bk_ref, w1g_ref,
                b1_ref, y_ref, out_ref, m_ref, acc_ref):
    nb = pl.program_id(0)
    enc = _encode_groups(x_ref, sf_ref, wkc_ref, wks_ref, bk_ref, w1g_ref,
                         b1_ref)
    c2 = jnp.sum(enc * enc, axis=1, keepdims=True)
    s = jnp.dot(enc.astype(jnp.bfloat16), qt_ref[...],
                preferred_element_type=jnp.float32) - _LOG2E * c2
    bm = jnp.max(s, axis=0, keepdims=True)
    m_prev = jnp.where(nb == 0, jnp.full_like(bm, -1e30), m_ref[0:1, :])
    m_new = jnp.maximum(m_prev, bm)
    e = jnp.exp2((s - m_new).astype(jnp.bfloat16))
    yrow = jnp.broadcast_to(y_ref[0], (16, NB))
    ridx = jax.lax.broadcasted_iota(jnp.int32, (16, NB), 0)
    ya = jnp.logical_or(ridx == yrow, ridx == N_CLASSES).astype(jnp.bfloat16)
    p = jnp.dot(ya, e, preferred_element_type=jnp.float32)
    scale = jnp.exp2(m_prev - m_new)
    acc_prev = jnp.where(nb == 0, jnp.zeros_like(acc_ref[...]), acc_ref[...])
    acc = acc_prev * scale + p
    m_ref[0:1, :] = m_new
    acc_ref[...] = acc

    @pl.when(nb == N_BLOCKS - 1)
    def _():
        denom = acc[N_CLASSES:N_CLASSES + 1, :]
        res = jnp.log(acc / denom + _EPS)
        ridx2 = jax.lax.broadcasted_iota(jnp.int32, res.shape, 0)
        out_ref[...] = jnp.where(ridx2 < N_CLASSES, res, 0.0)


def _fused(qt, x, sf, wkc, wks, bk, w1g, b1, y3):
    return pl.pallas_call(
        _fused_body,
        grid=(N_BLOCKS,),
        in_specs=[pl.BlockSpec((D_HIDDEN, B), lambda i: (0, 0)),
                  pl.BlockSpec((NB, N_NUM), lambda i: (i, 0))]
        + _ENC_IN_SPECS
        + [pl.BlockSpec((1, 1, NB), lambda i: (i, 0, 0))],
        out_specs=pl.BlockSpec((16, B), lambda i: (0, 0)),
        out_shape=jax.ShapeDtypeStruct((16, B), jnp.float32),
        scratch_shapes=[
            pltpu.VMEM((8, B), jnp.float32),
            pltpu.VMEM((16, B), jnp.float32),
        ],
        compiler_params=pltpu.CompilerParams(
            dimension_semantics=("arbitrary",)),
    )(qt, x, sf, wkc, wks, bk, w1g, b1, y3)


def kernel(x_num, candidate_x_num, candidate_y, freq, W_enc, b_enc, W1, b1):
    f32 = jnp.float32
    freq = freq.astype(f32)
    n_idx = jnp.arange(N_NUM)
    cols = (n_idx * N_FREQ)[:, None] + jnp.arange(N_FREQ)[None, :]
    sf = jnp.zeros((N_NUM, Z_COLS), f32)
    sf = sf.at[n_idx[:, None], cols].set(2.0 * freq)
    sf = sf.astype(jnp.bfloat16)

    we = W_enc.astype(f32).reshape(G, K_GRP, 2, N_FREQ, D_EMB)
    wkc = jnp.zeros((G, ZW, KOUT), f32)
    wks = jnp.zeros((G, ZW, KOUT), f32)
    for j in range(K_GRP):
        rr = j * N_FREQ
        cc = j * D_EMB
        wkc = wkc.at[:, rr:rr + N_FREQ, cc:cc + D_EMB].set(we[:, j, 0])
        wks = wks.at[:, rr:rr + N_FREQ, cc:cc + D_EMB].set(we[:, j, 1])
    wkc = wkc.astype(jnp.bfloat16)
    wks = wks.astype(jnp.bfloat16)

    bk = b_enc.astype(f32).reshape(G, 1, KOUT)
    w1g = W1.astype(f32).reshape(G, KOUT, D_HIDDEN).astype(jnp.bfloat16)
    b1r = b1.astype(f32).reshape(1, D_HIDDEN)

    qt = _encode_t(x_num.astype(f32), sf, wkc, wks, bk, w1g, b1r, B // 2)
    y3 = candidate_y.astype(jnp.int32).reshape(N_BLOCKS, 1, NB)
    out = _fused(qt, candidate_x_num.astype(f32), sf, wkc, wks, bk, w1g,
                 b1r, y3)
    return out[:N_CLASSES, :].T
